# baseline (speedup 1.0000x reference)
import functools

import jax
import jax.numpy as jnp
from jax import lax
from jax.experimental import pallas as pl
from jax.experimental.pallas import tpu as pltpu
from jax.experimental.pallas import tpu_sc as plsc

_NC = 2
_NS = 16
_NW = _NC * _NS
_CH = 128
_NRB = 4
_NSB = 2
_NIB = 4
_AHEAD = 2
_L16 = 16


@functools.partial(jax.jit, static_argnames=("seq_len", "emb"))
def _sc_embed(seq_t, table2, *, seq_len, emb):
    mesh = plsc.VectorSubcoreMesh(core_axis_name="c", subcore_axis_name="s")
    groups = _CH // _L16

    @functools.partial(
        pl.kernel,
        out_type=jax.ShapeDtypeStruct((seq_len, emb, _NW * _CH), jnp.float32),
        mesh=mesh,
        scratch_types=[
            pltpu.VMEM((seq_len, _CH), jnp.int32),
            pltpu.VMEM((_NIB, _CH), jnp.int32),
            *[pltpu.VMEM((_CH, 2 * emb), jnp.float32) for _ in range(_NRB)],
            *[pltpu.VMEM((emb, _CH), jnp.float32) for _ in range(_NSB)],
            *[pltpu.SemaphoreType.DMA for _ in range(_NRB + _NSB)],
        ],
        compiler_params=pltpu.CompilerParams(
            use_tc_tiling_on_sc=True, needs_layout_passes=False
        ),
    )
    def body(seq_hbm, table_hbm, out_hbm, idx_v, half_v, *bufs_and_sems):
        rows = bufs_and_sems[:_NRB]
        slabs = bufs_and_sems[_NRB:_NRB + _NSB]
        gsems = bufs_and_sems[_NRB + _NSB:2 * _NRB + _NSB]
        ssems = bufs_and_sems[2 * _NRB + _NSB:]
        wid = lax.axis_index("s") * _NC + lax.axis_index("c")
        col0 = wid * _CH

        def fire_gather(l, rb, ib):
            for t in range(groups):
                sl = pl.ds(t * _L16, _L16)
                half_v[ib, sl] = jax.lax.shift_right_logical(idx_v[l, sl], 1)
            pltpu.make_async_copy(
                table_hbm.at[half_v.at[ib]], rows[rb], gsems[rb]
            ).start()

        def wait_gather(rb):
            pltpu.make_async_copy(
                table_hbm.at[half_v.at[0]], rows[rb], gsems[rb]
            ).wait()

        def store_desc(l, sb):
            return pltpu.make_async_copy(
                slabs[sb], out_hbm.at[l, :, pl.ds(col0, _CH)], ssems[sb]
            )

        pltpu.sync_copy(seq_hbm.at[:, pl.ds(col0, _CH)], idx_v)
        for p in range(_AHEAD):
            fire_gather(p, p % _NRB, p % _NIB)

        @pl.loop(0, seq_len, step=_NRB)
        def _(l0):
            for k in range(_NRB):
                l = l0 + k
                rb = k % _NRB
                sb = k % _NSB

                @pl.when(l + _AHEAD < seq_len)
                def _():
                    fire_gather(
                        l + _AHEAD, (k + _AHEAD) % _NRB, (k + _AHEAD) % _NIB
                    )

                wait_gather(rb)

                @pl.when(l >= _NSB)
                def _():
                    store_desc(0, sb).wait()

                @pl.loop(0, groups)
                def _(t):
                    tok = pl.ds(t * _L16, _L16)
                    par = jax.lax.shift_left(
                        jnp.bitwise_and(idx_v[l, tok], 1), 6
                    )
                    row_ids = jax.lax.iota(jnp.int32, _L16) + t * _L16
                    for e in range(emb):
                        vals = plsc.load_gather(rows[rb], [row_ids, par + e])
                        slabs[sb][e, tok] = vals

                store_desc(l, sb).start()

        for b in range(_NSB):
            store_desc(0, b).wait()

    return body(seq_t, table2)


def kernel(sequences, embedding_weight):
    b, l = sequences.shape
    v, emb = embedding_weight.shape
    seq_t = sequences.T.astype(jnp.int32)
    table2 = embedding_weight.reshape(v // 2, 2 * emb)
    out_t = _sc_embed(seq_t, table2, seq_len=l, emb=emb)
    return out_t.transpose(2, 0, 1)
